# baseline (speedup 1.0000x reference)
import jax
import jax.numpy as jnp
from jax import lax
from jax.experimental import pallas as pl
from jax.experimental.pallas import tpu as pltpu
from jax.experimental.pallas import tpu_sc as plsc

M = 1000000
D = 59
DP = 64
B = 262144

NC = 2
NT = 16
L = 16

NSEG = 40
SEG = M // NSEG
NPASS = NSEG // NC
TPT = 1568
LAST_START = SEG - TPT

IDXSL = B // NT
CHUNKS = IDXSL // L

FL = 128
BUF = FL + L
TRASH = BUF
BUFA = BUF + L


def _body(mem, val, idx, out, seg, idxv, lbuf, jbuf, lfl, jfl, valbuf, sem):
    c = lax.axis_index("c")
    t = lax.axis_index("s")
    iota = lax.iota(jnp.int32, L)

    lpad = iota + SEG
    jpad = iota

    pltpu.sync_copy(idx.at[pl.ds(t * IDXSL, IDXSL)], idxv)

    start_t = jnp.minimum(t * TPT, LAST_START)

    def reset_bufs():
        for k in range(BUF // L):
            lbuf[pl.ds(k * L, L)] = lpad
            jbuf[pl.ds(k * L, L)] = jpad

    def flush():
        for k in range(FL // L):
            lfl[pl.ds(k * L, L)] = lbuf[pl.ds(k * L, L)]
            jfl[pl.ds(k * L, L)] = jbuf[pl.ds(k * L, L)]
        pltpu.async_copy(val.at[jfl], valbuf, sem).wait()
        pltpu.sync_copy(valbuf, seg.at[lfl], add=True)

    def pass_body(p, _):
        base = (c * NPASS + p) * SEG
        row0 = base + start_t

        pltpu.sync_copy(mem.at[pl.ds(row0, TPT)], seg.at[pl.ds(start_t, TPT)])
        plsc.subcore_barrier()

        reset_bufs()

        def scan_body(i, fill):
            v = idxv[pl.ds(i * L, L)]
            in_seg = (v >= base) & (v < base + SEG)
            lidx = v - base
            jvec = iota + (t * IDXSL + i * L)
            inc = in_seg.astype(jnp.int32)
            excl = plsc.cumsum(inc) - inc
            dest = jnp.where(in_seg, fill + excl, TRASH + iota)
            plsc.store_scatter(lbuf, [dest], lidx)
            plsc.store_scatter(jbuf, [dest], jvec)
            fill = fill + jnp.sum(inc)
            full = fill >= FL

            @pl.when(full)
            def _():
                flush()
                nf = fill - FL
                lv_l = lbuf[pl.ds(FL, L)]
                lv_j = jbuf[pl.ds(FL, L)]
                reset_bufs()
                keep = iota < nf
                kdest = jnp.where(keep, iota, TRASH + iota)
                plsc.store_scatter(lbuf, [kdest], lv_l)
                plsc.store_scatter(jbuf, [kdest], lv_j)
                plsc.store_scatter(lbuf, [nf + iota], lpad)
                plsc.store_scatter(jbuf, [nf + iota], jpad)

            return jnp.where(full, fill - FL, fill)

        fill = lax.fori_loop(0, CHUNKS, scan_body, jnp.int32(0))

        plsc.store_scatter(lbuf, [fill + iota], lpad)
        plsc.store_scatter(jbuf, [fill + iota], jpad)

        @pl.when(fill > 0)
        def _():
            flush()

        plsc.subcore_barrier()
        pltpu.sync_copy(seg.at[pl.ds(start_t, TPT)], out.at[pl.ds(row0, TPT)])
        plsc.subcore_barrier()
        return 0

    lax.fori_loop(0, NPASS, pass_body, 0)


@jax.jit
def _scatter_add(memp, valp, idx):
    mesh = plsc.VectorSubcoreMesh(
        core_axis_name="c", subcore_axis_name="s", num_cores=NC, num_subcores=NT
    )
    return pl.kernel(
        _body,
        out_type=jax.ShapeDtypeStruct((M, DP), jnp.float32),
        mesh=mesh,
        compiler_params=pltpu.CompilerParams(
            needs_layout_passes=False, use_tc_tiling_on_sc=False
        ),
        scratch_types=[
            pltpu.VMEM_SHARED((SEG + L, DP), jnp.float32),
            pltpu.VMEM((IDXSL,), jnp.int32),
            pltpu.VMEM((BUFA,), jnp.int32),
            pltpu.VMEM((BUFA,), jnp.int32),
            pltpu.VMEM((FL,), jnp.int32),
            pltpu.VMEM((FL,), jnp.int32),
            pltpu.VMEM((FL, DP), jnp.float32),
            pltpu.SemaphoreType.DMA,
        ],
    )(memp, valp, idx)


def kernel(mem, val, idx):
    memp = jnp.pad(mem, ((0, 0), (0, DP - D)))
    valp = jnp.pad(val, ((0, 0), (0, DP - D)))
    outp = _scatter_add(memp, valp, idx)
    return outp[:, :D]
